# baseline (speedup 1.0000x reference)
import jax
import jax.numpy as jnp
import numpy as np
from jax import lax
from jax.experimental import pallas as pl
from jax.experimental.pallas import tpu as pltpu
from jax.experimental.pallas import tpu_sc as plsc

_D = 128
_OUT = 12

_SEL1 = np.array([0, 0, 0, 1, 1, 1, 2, 2, 2, 3, 3, 3, 7, 7, 7, 7],
                 dtype=np.int32)
_SEL2 = np.array([4, 5, 6, 4, 5, 6, 4, 5, 6, 4, 5, 6, 7, 7, 7, 7],
                 dtype=np.int32)


def _take16(v, sel):
    dnums = lax.GatherDimensionNumbers(
        offset_dims=(), collapsed_slice_dims=(0,), start_index_map=(0,))
    return lax.gather(v, sel[:, None], dnums, (1,),
                      mode=lax.GatherScatterMode.PROMISE_IN_BOUNDS)


def _body(pack_hbm, tab_hbm, out_hbm, pack_v, rows_v, sem):
    pltpu.sync_copy(pack_hbm, pack_v)
    v3 = pack_v[pl.ds(0, 16)]
    hi = pack_v[pl.ds(16, 16)]
    v1 = _take16(hi, pack_v[pl.ds(32, 16)])
    v2 = _take16(hi, pack_v[pl.ds(48, 16)])
    idx = v1 * 4096 + v2 * 64 + v3
    pltpu.async_copy(tab_hbm.at[idx], rows_v, sem).wait()
    d0 = pltpu.async_copy(rows_v.at[pl.ds(0, 3)], out_hbm.at[np.int32(0)], sem)
    d1 = pltpu.async_copy(rows_v.at[pl.ds(3, 3)], out_hbm.at[np.int32(1)], sem)
    d2 = pltpu.async_copy(rows_v.at[pl.ds(6, 3)], out_hbm.at[np.int32(2)], sem)
    d3 = pltpu.async_copy(rows_v.at[pl.ds(9, 3)], out_hbm.at[np.int32(3)], sem)
    d0.wait()
    d1.wait()
    d2.wait()
    d3.wait()


def _gather12(pack, tab):
    mesh = plsc.VectorSubcoreMesh(core_axis_name="c", subcore_axis_name="s",
                                  num_cores=1, num_subcores=1)
    f = pl.kernel(
        _body,
        mesh=mesh,
        out_type=jax.ShapeDtypeStruct((4, 3, _D), jnp.float32),
        scratch_types=[
            pltpu.VMEM((64,), jnp.int32),
            pltpu.VMEM((16, _D), jnp.float32),
            pltpu.SemaphoreType.DMA,
        ],
    )
    return f(pack, tab)


def kernel(x, index1, index2, index3):
    tab = x.reshape(-1, _D)
    zpad = jnp.zeros((4,), index3.dtype)
    cat = jnp.concatenate([
        index3.reshape(-1), zpad,
        index1.reshape(-1), index2.reshape(-1),
        zpad, zpad, jnp.zeros((1,), index3.dtype),
    ])
    pack = jnp.concatenate([
        cat.astype(jnp.int32), jnp.asarray(_SEL1), jnp.asarray(_SEL2),
    ])
    return _gather12(pack, tab)


# speedup vs baseline: 1.0924x; 1.0924x over previous
import jax
import jax.numpy as jnp
import numpy as np
from jax import lax
from jax.experimental import pallas as pl
from jax.experimental.pallas import tpu as pltpu
from jax.experimental.pallas import tpu_sc as plsc

_D = 128
_OUT = 12


def _body(pack_hbm, tab_hbm, out_hbm, smem, sem):
    pltpu.sync_copy(pack_hbm, smem)
    descs = []
    for k in range(_OUT):
        i = k // 3
        j = k % 3
        idx = smem[12 + i] * 4096 + smem[16 + j] * 64 + smem[k]
        descs.append(pltpu.async_copy(
            tab_hbm.at[pl.ds(idx, 1)], out_hbm.at[pl.ds(np.int32(k), 1)], sem))
    for d in descs:
        d.wait()


def _gather12(pack, tab):
    mesh = plsc.ScalarSubcoreMesh(axis_name="c", num_cores=1)
    f = pl.kernel(
        _body,
        mesh=mesh,
        out_type=jax.ShapeDtypeStruct((_OUT, _D), jnp.float32),
        scratch_types=[
            pltpu.SMEM((32,), jnp.int32),
            pltpu.SemaphoreType.DMA,
        ],
    )
    return f(pack, tab)


def kernel(x, index1, index2, index3):
    tab = x.reshape(-1, _D)
    zpad = jnp.zeros((4,), index3.dtype)
    cat = jnp.concatenate([
        index3.reshape(-1),
        index1.reshape(-1), index2.reshape(-1),
        zpad, zpad, zpad, jnp.zeros((1,), index3.dtype),
    ])
    pack = cat.astype(jnp.int32)
    out = _gather12(pack, tab)
    return out.reshape(4, 3, _D)
